# Initial kernel scaffold; baseline (speedup 1.0000x reference)
import jax
import jax.numpy as jnp
from jax.experimental import pallas as pl


def kernel(input, another_input, W1, W2):
    raise NotImplementedError("write your pallas kernel here")


# speedup vs baseline: 1.4671x; 1.4671x over previous
import functools

import jax
import jax.numpy as jnp
from jax import lax
from jax.experimental import pallas as pl
from jax.experimental.pallas import tpu as pltpu
from jax.experimental.pallas import tpu_sc as plsc

_NW = 32
_CHUNK = 128
_D = 32


@functools.partial(jax.jit, static_argnums=(4,))
def _run(idx1, idx2, w1, w2, n_chunks_per_w):
    total_chunks = _NW * n_chunks_per_w
    mesh = plsc.VectorSubcoreMesh(core_axis_name="c", subcore_axis_name="s")

    @functools.partial(
        pl.kernel,
        mesh=mesh,
        out_type=jax.ShapeDtypeStruct((total_chunks, _CHUNK, _D), jnp.float32),
        compiler_params=pltpu.CompilerParams(use_tc_tiling_on_sc=False),
        scratch_types=[
            pltpu.VMEM((n_chunks_per_w, _CHUNK), jnp.int32),
            pltpu.VMEM((n_chunks_per_w, _CHUNK), jnp.int32),
            pltpu.VMEM((_CHUNK, _D), jnp.float32),
            pltpu.VMEM((_CHUNK, _D), jnp.float32),
            pltpu.SemaphoreType.DMA,
            pltpu.SemaphoreType.DMA,
        ],
    )
    def k(idx1_hbm, idx2_hbm, w1_hbm, w2_hbm, out_hbm,
          i1_v, i2_v, buf_a, buf_b, sem_a, sem_b):
        cid = lax.axis_index("c")
        sid = lax.axis_index("s")
        wid = sid * 2 + cid
        pltpu.sync_copy(idx1_hbm.at[wid], i1_v)
        pltpu.sync_copy(idx2_hbm.at[wid], i2_v)

        def body(c, carry):
            gc = wid * n_chunks_per_w + c
            cp_a = pltpu.async_copy(w1_hbm.at[i1_v.at[c]], buf_a, sem_a)
            cp_b = pltpu.async_copy(w2_hbm.at[i2_v.at[c]], buf_b, sem_b)
            cp_a.wait()
            cp_b.wait()

            def add_row(r, carry2):
                buf_a[r, pl.ds(0, 16)] = buf_a[r, pl.ds(0, 16)] + buf_b[r, pl.ds(0, 16)]
                buf_a[r, pl.ds(16, 16)] = buf_a[r, pl.ds(16, 16)] + buf_b[r, pl.ds(16, 16)]
                return carry2

            lax.fori_loop(0, _CHUNK, add_row, 0)
            pltpu.sync_copy(buf_a, out_hbm.at[gc])
            return carry

        lax.fori_loop(0, n_chunks_per_w, body, 0)

    return k(idx1, idx2, w1, w2)


def kernel(input, another_input, W1, W2):
    B, H = input.shape
    total = B * H
    n_chunks_per_w = total // (_NW * _CHUNK)
    idx1 = input.reshape(_NW, n_chunks_per_w, _CHUNK).astype(jnp.int32)
    idx2 = another_input.reshape(_NW, n_chunks_per_w, _CHUNK).astype(jnp.int32)
    out = _run(idx1, idx2, W1, W2, n_chunks_per_w)
    return out.reshape(B, H, _D)
